# Initial kernel scaffold; baseline (speedup 1.0000x reference)
import jax
import jax.numpy as jnp
from jax.experimental import pallas as pl


def kernel(input, target):
    raise NotImplementedError("write your pallas kernel here")


# speedup vs baseline: 1.2294x; 1.2294x over previous
import functools

import jax
import jax.numpy as jnp
from jax import lax
from jax.experimental import pallas as pl
from jax.experimental.pallas import tpu as pltpu
from jax.experimental.pallas import tpu_sc as plsc

G, B, V = 26, 4096, 1000
ROWS = G * B
NW = 32
RPW = ROWS // NW
RCHUNK = 16
NCHUNK = RPW // RCHUNK
UNROLL = 4
NSTEP = V // UNROLL
NEG = -3.0e38

_mesh = plsc.VectorSubcoreMesh(core_axis_name="c", subcore_axis_name="s")


@functools.partial(
    pl.kernel,
    out_type=jax.ShapeDtypeStruct((NW, 16), jnp.int32),
    mesh=_mesh,
    compiler_params=pltpu.CompilerParams(needs_layout_passes=False),
    scratch_types=[
        pltpu.VMEM((RCHUNK * V,), jnp.float32),
        pltpu.VMEM((RPW,), jnp.int32),
        pltpu.VMEM((16,), jnp.int32),
    ],
)
def _sc_count(in_hbm, tgt_hbm, out_hbm, buf, tgts, outv):
    wid = lax.axis_index("s") * 2 + lax.axis_index("c")
    row0 = wid * RPW
    pltpu.sync_copy(tgt_hbm.at[pl.ds(row0, RPW)], tgts)
    lanes = lax.iota(jnp.int32, 16)
    rowbase = lanes * V

    def chunk_body(c, cnt):
        base = (row0 + c * RCHUNK) * V
        pltpu.sync_copy(in_hbm.at[pl.ds(base, RCHUNK * V)], buf)

        def col_body(jj, carry):
            ms = list(carry[:UNROLL])
            ivs = list(carry[UNROLL:2 * UNROLL])
            bs = list(carry[2 * UNROLL:])
            for a in range(UNROLL):
                v = plsc.load_gather(buf, [ivs[a]])
                gt = v > ms[a]
                ms[a] = jnp.where(gt, v, ms[a])
                bs[a] = jnp.where(gt, ivs[a], bs[a])
                ivs[a] = ivs[a] + UNROLL
            return tuple(ms) + tuple(ivs) + tuple(bs)

        init = (
            tuple(jnp.full((16,), NEG, jnp.float32) for _ in range(UNROLL))
            + tuple(rowbase + a for a in range(UNROLL))
            + tuple(jnp.full((16,), jnp.int32(0)) for _ in range(UNROLL))
        )
        res = lax.fori_loop(0, NSTEP, col_body, init)
        ms = res[:UNROLL]
        bs = res[2 * UNROLL:]

        def merge(ma, ba, mb, bb):
            take_b = (mb > ma) | ((mb == ma) & (bb < ba))
            return jnp.where(take_b, mb, ma), jnp.where(take_b, bb, ba)

        m01, b01 = merge(ms[0], bs[0], ms[1], bs[1])
        m23, b23 = merge(ms[2], bs[2], ms[3], bs[3])
        _, bfin = merge(m01, b01, m23, b23)
        cols = bfin - rowbase
        tgt_vec = tgts[pl.ds(c * RCHUNK, 16)]
        return cnt + jnp.where(cols != tgt_vec, jnp.int32(1), jnp.int32(0))

    cnt = lax.fori_loop(0, NCHUNK, chunk_body, jnp.full((16,), jnp.int32(0)))
    outv[...] = cnt
    pltpu.sync_copy(outv, out_hbm.at[wid])


def kernel(input, target):
    flat = input.reshape(-1)
    tgt = jnp.asarray(target, jnp.int32).T.reshape(-1)
    partials = _sc_count(flat, tgt)
    return jnp.sum(partials).astype(jnp.int32)
